# baseline (speedup 1.0000x reference)
import functools

import jax
import jax.numpy as jnp
from jax import lax
from jax.experimental import pallas as pl
from jax.experimental.pallas import tpu as pltpu
from jax.experimental.pallas import tpu_sc as plsc

N = 10000
E = 160000
H = 256
G = 64
NP = 10240
EP = 163840
EROWS = 1280
TILES = 32
RPT = EROWS // TILES
NSLICE = NP // 16


BLK = 2560
NBLK = 4


def _k1_body(x_ref, w2t_ref, out_ref):
    out_ref[...] = lax.dot_general(
        w2t_ref[...], x_ref[...],
        dimension_numbers=(((1,), (1,)), ((), ())),
        preferred_element_type=jnp.float32,
        precision=lax.Precision.DEFAULT,
    )


def _project(x, w2t):
    return pl.pallas_call(
        _k1_body,
        grid=(NBLK,),
        in_specs=[
            pl.BlockSpec((BLK, H), lambda i: (i, 0)),
            pl.BlockSpec((2, H), lambda i: (0, 0)),
        ],
        out_specs=pl.BlockSpec((2, BLK), lambda i: (0, i)),
        out_shape=jax.ShapeDtypeStruct((2, N), jnp.float32),
    )(x, w2t)


def _sc_body(y2t_hbm, src_hbm, dst_hbm, dummy_hbm, out_hbm,
             yrel_v, src_v, dst_v, vals_v, zero_v, agg_sh, sem_in, sem_sc):
    cid = lax.axis_index("c")
    sid = lax.axis_index("s")
    tid = cid * 16 + sid

    in1 = pltpu.async_copy(y2t_hbm.at[0], yrel_v, sem_in)
    in2 = pltpu.async_copy(src_hbm.at[pl.ds(tid * RPT, RPT)], src_v, sem_in)
    in3 = pltpu.async_copy(dst_hbm.at[pl.ds(tid * RPT, RPT)], dst_v, sem_in)

    zeros16 = jnp.zeros((16,), jnp.float32)
    for k in range(NSLICE // 16):
        zero_v[pl.ds(k * 16, 16)] = zeros16
    pltpu.sync_copy(zero_v, agg_sh.at[pl.ds(sid * NSLICE, NSLICE)])
    in1.wait()
    in2.wait()
    in3.wait()
    plsc.subcore_barrier()

    @plsc.parallel_loop(0, RPT, unroll=4)
    def edge_row(j):
        for k in range(8):
            idx = src_v[j, pl.ds(k * 16, 16)]
            vals_v[j, pl.ds(k * 16, 16)] = plsc.load_gather(yrel_v, [idx])
        pltpu.async_copy(vals_v.at[j], agg_sh.at[dst_v.at[j]], sem_sc,
                         add=True)
    pltpu.make_async_copy(dummy_hbm, vals_v, sem_sc).wait()
    plsc.subcore_barrier()

    pltpu.sync_copy(agg_sh.at[pl.ds(sid * NSLICE, NSLICE)],
                    out_hbm.at[cid, pl.ds(sid * NSLICE, NSLICE)])


def _edge_agg(y2t, src_r, dst_r):
    mesh = plsc.VectorSubcoreMesh(core_axis_name="c", subcore_axis_name="s")
    k = functools.partial(
        pl.kernel,
        out_type=jax.ShapeDtypeStruct((2, NP), jnp.float32),
        mesh=mesh,
        compiler_params=pltpu.CompilerParams(needs_layout_passes=False),
        scratch_types=[
            pltpu.VMEM((N,), jnp.float32),
            pltpu.VMEM((RPT, 128), jnp.int32),
            pltpu.VMEM((RPT, 128), jnp.int32),
            pltpu.VMEM((RPT, 128), jnp.float32),
            pltpu.VMEM((NSLICE,), jnp.float32),
            pltpu.VMEM_SHARED((NP,), jnp.float32),
            pltpu.SemaphoreType.DMA,
            pltpu.SemaphoreType.DMA,
        ],
    )(_sc_body)
    return k(y2t, src_r, dst_r, jnp.zeros((RPT, 128), jnp.float32))


def _k3_body(x_ref, y2tf_ref, aggf_ref, batchf_ref, y2t_ref, agg_ref,
             batch_ref, brel_ref, out_ref, m_ref, l_ref, acc_ref):
    i = pl.program_id(0)
    neg_inf = jnp.float32(-jnp.inf)

    @pl.when(i == 0)
    def _():
        xconv = (aggf_ref[0:1, :N] + aggf_ref[1:2, :N] + y2tf_ref[1:2, :]
                 + brel_ref[0, 0])
        gids = lax.broadcasted_iota(jnp.int32, (G, N), 0)
        mask = gids == batchf_ref[...]
        m = jnp.max(jnp.where(mask, xconv, neg_inf), axis=1, keepdims=True)
        ex = jnp.exp(jnp.where(mask, xconv - m, neg_inf))
        m_ref[...] = m
        l_ref[...] = jnp.sum(ex, axis=1, keepdims=True)

    @pl.when(i > 0)
    def _():
        b = i - 1
        xconv = (agg_ref[0:1, :] + agg_ref[1:2, :] + y2t_ref[1:2, :]
                 + brel_ref[0, 0])
        gids = lax.broadcasted_iota(jnp.int32, (G, BLK), 0)
        node = b * BLK + lax.broadcasted_iota(jnp.int32, (G, BLK), 1)
        mask = (gids == batch_ref[...]) & (node < N)

        ex = jnp.exp(jnp.where(mask, xconv - m_ref[...], neg_inf))
        p = ex / (l_ref[...] + jnp.float32(1e-16))
        node_col = b * BLK + lax.broadcasted_iota(jnp.int32, (BLK, 1), 0)
        xb = jnp.where(node_col < N, x_ref[...], jnp.float32(0))
        part = jnp.dot(p, xb, preferred_element_type=jnp.float32,
                       precision=lax.Precision.DEFAULT)
        acc_ref[...] = jnp.where(b == 0, part, acc_ref[...] + part)

    @pl.when(i == NBLK)
    def _():
        out_ref[...] = acc_ref[...]


def _pool(x, y2t, agg2, batch_r, brel):
    blk_i = lambda i: (0, jnp.maximum(i - 1, 0))
    return pl.pallas_call(
        _k3_body,
        grid=(NBLK + 1,),
        in_specs=[
            pl.BlockSpec((BLK, H), lambda i: (jnp.maximum(i - 1, 0), 0)),
            pl.BlockSpec((2, N), lambda i: (0, 0)),
            pl.BlockSpec((2, NP), lambda i: (0, 0)),
            pl.BlockSpec((1, N), lambda i: (0, 0)),
            pl.BlockSpec((2, BLK), blk_i),
            pl.BlockSpec((2, BLK), blk_i),
            pl.BlockSpec((1, BLK), blk_i),
            pl.BlockSpec((1, 1), lambda i: (0, 0)),
        ],
        out_specs=pl.BlockSpec((G, H), lambda i: (0, 0)),
        out_shape=jax.ShapeDtypeStruct((G, H), jnp.float32),
        scratch_shapes=[
            pltpu.VMEM((G, 1), jnp.float32),
            pltpu.VMEM((G, 1), jnp.float32),
            pltpu.VMEM((G, H), jnp.float32),
        ],
    )(x, y2t, agg2, batch_r, y2t, agg2, batch_r, brel)


def kernel(x, edge_index, batch, W_rel, b_rel, W_root):
    w2t = jnp.concatenate([W_rel, W_root], axis=1).T.astype(jnp.float32)
    y2t = _project(x, w2t)

    src = edge_index[0].astype(jnp.int32)
    dst = edge_index[1].astype(jnp.int32)
    npad = EP - E
    src_r = jnp.concatenate([src, jnp.zeros((npad,), jnp.int32)]).reshape(EROWS, 128)
    dst_r = jnp.concatenate(
        [dst, N + (jnp.arange(npad, dtype=jnp.int32) % (NP - N))]).reshape(EROWS, 128)
    agg2 = _edge_agg(y2t, src_r, dst_r)

    batch_r = batch.astype(jnp.int32).reshape(1, N)
    brel = b_rel.reshape(1, 1).astype(jnp.float32)
    return _pool(x, y2t, agg2, batch_r, brel)


# speedup vs baseline: 1.0079x; 1.0079x over previous
import functools

import jax
import jax.numpy as jnp
from jax import lax
from jax.experimental import pallas as pl
from jax.experimental.pallas import tpu as pltpu
from jax.experimental.pallas import tpu_sc as plsc

N = 10000
E = 160000
H = 256
G = 64
NP = 10240
EP = 163840
EROWS = 1280
TILES = 32
RPT = EROWS // TILES
NSLICE = NP // 16


BLK = 2560
NBLK = 4


def _k1_body(x_ref, w2t_ref, out_ref):
    out_ref[...] = lax.dot_general(
        w2t_ref[...], x_ref[...],
        dimension_numbers=(((1,), (1,)), ((), ())),
        preferred_element_type=jnp.float32,
        precision=lax.Precision.DEFAULT,
    )


def _project(x, w2t):
    return pl.pallas_call(
        _k1_body,
        grid=(NBLK,),
        in_specs=[
            pl.BlockSpec((BLK, H), lambda i: (i, 0)),
            pl.BlockSpec((2, H), lambda i: (0, 0)),
        ],
        out_specs=pl.BlockSpec((2, BLK), lambda i: (0, i)),
        out_shape=jax.ShapeDtypeStruct((2, N), jnp.float32),
    )(x, w2t)


def _sc_body(y2t_hbm, src_hbm, dst_hbm, dummy_hbm, out_hbm,
             yrel_v, src_v, dst_v, vals_v, zero_v, agg_sh, sem_in, sem_sc):
    cid = lax.axis_index("c")
    sid = lax.axis_index("s")
    tid = cid * 16 + sid

    in1 = pltpu.async_copy(y2t_hbm.at[0], yrel_v, sem_in)
    in2 = pltpu.async_copy(src_hbm.at[pl.ds(tid * RPT, RPT)], src_v, sem_in)
    in3 = pltpu.async_copy(dst_hbm.at[pl.ds(tid * RPT, RPT)], dst_v, sem_in)

    zeros16 = jnp.zeros((16,), jnp.float32)
    for k in range(NSLICE // 16):
        zero_v[pl.ds(k * 16, 16)] = zeros16
    pltpu.sync_copy(zero_v, agg_sh.at[pl.ds(sid * NSLICE, NSLICE)])
    in1.wait()
    in2.wait()
    in3.wait()
    plsc.subcore_barrier()

    def edge_row(j, carry):
        for k in range(8):
            idx = src_v[j, pl.ds(k * 16, 16)]
            vals_v[j, pl.ds(k * 16, 16)] = plsc.load_gather(yrel_v, [idx])
        pltpu.async_copy(vals_v.at[j], agg_sh.at[dst_v.at[j]], sem_sc,
                         add=True)
        return carry

    lax.fori_loop(0, RPT, edge_row, 0)
    pltpu.make_async_copy(dummy_hbm, vals_v, sem_sc).wait()
    plsc.subcore_barrier()

    pltpu.sync_copy(agg_sh.at[pl.ds(sid * NSLICE, NSLICE)],
                    out_hbm.at[cid, pl.ds(sid * NSLICE, NSLICE)])


def _edge_agg(y2t, src_r, dst_r):
    mesh = plsc.VectorSubcoreMesh(core_axis_name="c", subcore_axis_name="s")
    k = functools.partial(
        pl.kernel,
        out_type=jax.ShapeDtypeStruct((2, NP), jnp.float32),
        mesh=mesh,
        compiler_params=pltpu.CompilerParams(needs_layout_passes=False),
        scratch_types=[
            pltpu.VMEM((N,), jnp.float32),
            pltpu.VMEM((RPT, 128), jnp.int32),
            pltpu.VMEM((RPT, 128), jnp.int32),
            pltpu.VMEM((RPT, 128), jnp.float32),
            pltpu.VMEM((NSLICE,), jnp.float32),
            pltpu.VMEM_SHARED((NP,), jnp.float32),
            pltpu.SemaphoreType.DMA,
            pltpu.SemaphoreType.DMA,
        ],
    )(_sc_body)
    return k(y2t, src_r, dst_r, jnp.zeros((RPT, 128), jnp.float32))


def _k3_body(x_ref, y2tf_ref, aggf_ref, batchf_ref, y2t_ref, agg_ref,
             batch_ref, brel_ref, out_ref, m_ref, l_ref, acc_ref):
    i = pl.program_id(0)
    neg_inf = jnp.float32(-jnp.inf)

    @pl.when(i == 0)
    def _():
        xconv = (aggf_ref[0:1, :N] + aggf_ref[1:2, :N] + y2tf_ref[1:2, :]
                 + brel_ref[0, 0])
        gids = lax.broadcasted_iota(jnp.int32, (G, N), 0)
        mask = gids == batchf_ref[...]
        m = jnp.max(jnp.where(mask, xconv, neg_inf), axis=1, keepdims=True)
        ex = jnp.exp(jnp.where(mask, xconv - m, neg_inf))
        m_ref[...] = m
        l_ref[...] = jnp.sum(ex, axis=1, keepdims=True)

    @pl.when(i > 0)
    def _():
        b = i - 1
        xconv = (agg_ref[0:1, :] + agg_ref[1:2, :] + y2t_ref[1:2, :]
                 + brel_ref[0, 0])
        gids = lax.broadcasted_iota(jnp.int32, (G, BLK), 0)
        node = b * BLK + lax.broadcasted_iota(jnp.int32, (G, BLK), 1)
        mask = (gids == batch_ref[...]) & (node < N)

        ex = jnp.exp(jnp.where(mask, xconv - m_ref[...], neg_inf))
        p = ex / (l_ref[...] + jnp.float32(1e-16))
        node_col = b * BLK + lax.broadcasted_iota(jnp.int32, (BLK, 1), 0)
        xb = jnp.where(node_col < N, x_ref[...], jnp.float32(0))
        part = jnp.dot(p, xb, preferred_element_type=jnp.float32,
                       precision=lax.Precision.DEFAULT)
        acc_ref[...] = jnp.where(b == 0, part, acc_ref[...] + part)

    @pl.when(i == NBLK)
    def _():
        out_ref[...] = acc_ref[...]


def _pool(x, y2t, agg2, batch_r, brel):
    blk_i = lambda i: (0, jnp.maximum(i - 1, 0))
    return pl.pallas_call(
        _k3_body,
        grid=(NBLK + 1,),
        in_specs=[
            pl.BlockSpec((BLK, H), lambda i: (jnp.maximum(i - 1, 0), 0)),
            pl.BlockSpec((2, N), lambda i: (0, 0)),
            pl.BlockSpec((2, NP), lambda i: (0, 0)),
            pl.BlockSpec((1, N), lambda i: (0, 0)),
            pl.BlockSpec((2, BLK), blk_i),
            pl.BlockSpec((2, BLK), blk_i),
            pl.BlockSpec((1, BLK), blk_i),
            pl.BlockSpec((1, 1), lambda i: (0, 0)),
        ],
        out_specs=pl.BlockSpec((G, H), lambda i: (0, 0)),
        out_shape=jax.ShapeDtypeStruct((G, H), jnp.float32),
        scratch_shapes=[
            pltpu.VMEM((G, 1), jnp.float32),
            pltpu.VMEM((G, 1), jnp.float32),
            pltpu.VMEM((G, H), jnp.float32),
        ],
    )(x, y2t, agg2, batch_r, y2t, agg2, batch_r, brel)


def kernel(x, edge_index, batch, W_rel, b_rel, W_root):
    w2t = jnp.concatenate([W_rel, W_root], axis=1).T.astype(jnp.float32)
    y2t = _project(x, w2t)

    src = edge_index[0].astype(jnp.int32)
    dst = edge_index[1].astype(jnp.int32)
    npad = EP - E
    src_r = jnp.concatenate([src, jnp.zeros((npad,), jnp.int32)]).reshape(EROWS, 128)
    dst_r = jnp.concatenate(
        [dst, N + (jnp.arange(npad, dtype=jnp.int32) % (NP - N))]).reshape(EROWS, 128)
    agg2 = _edge_agg(y2t, src_r, dst_r)

    batch_r = batch.astype(jnp.int32).reshape(1, N)
    brel = b_rel.reshape(1, 1).astype(jnp.float32)
    return _pool(x, y2t, agg2, batch_r, brel)
